# speedup vs baseline: 2.1975x; 2.1975x over previous
import functools

import jax
import jax.numpy as jnp
from jax import lax
from jax.experimental import pallas as pl
from jax.experimental.pallas import tpu as pltpu
from jax.experimental.pallas import tpu_sc as plsc

NC = 2
NS = 16
L = 16
NW = NC * NS

BATCH = 16384
BPW = BATCH // NW
CHUNK = 128
NCHUNK = BPW // CHUNK

_mesh = plsc.VectorSubcoreMesh(
    core_axis_name="c", subcore_axis_name="s", num_cores=NC, num_subcores=NS
)


@functools.partial(
    pl.kernel,
    out_type=jax.ShapeDtypeStruct((BATCH,), jnp.float32),
    mesh=_mesh,
    scratch_types=[
        pltpu.VMEM((NCHUNK, CHUNK), jnp.int32),
        pltpu.VMEM((NCHUNK, CHUNK), jnp.int32),
        pltpu.VMEM((NCHUNK, CHUNK), jnp.int32),
        pltpu.VMEM((BPW,), jnp.float32),
        pltpu.VMEM((BPW,), jnp.float32),
        pltpu.VMEM((BPW,), jnp.float32),
        pltpu.VMEM((BPW,), jnp.float32),
        pltpu.SemaphoreType.DMA,
    ],
)
def _three_bias_sum(users_hbm, items_hbm, values_hbm, ub_hbm, ib_hbm, db_hbm,
                    out_hbm, uidx, iidx, vidx, uval, ival, dval, out_v, sem):
    wid = lax.axis_index("s") * NC + lax.axis_index("c")
    base = wid * BPW

    row = wid * NCHUNK
    c_u = pltpu.async_copy(users_hbm.at[pl.ds(row, NCHUNK)], uidx, sem)
    c_i = pltpu.async_copy(items_hbm.at[pl.ds(row, NCHUNK)], iidx, sem)
    c_v = pltpu.async_copy(values_hbm.at[pl.ds(row, NCHUNK)], vidx, sem)
    c_u.wait()
    c_i.wait()
    c_v.wait()

    copies = []
    for j in range(NCHUNK):
        dst = pl.ds(j * CHUNK, CHUNK)
        copies.append(pltpu.async_copy(ub_hbm.at[uidx.at[j]], uval.at[dst], sem))
        copies.append(pltpu.async_copy(ib_hbm.at[iidx.at[j]], ival.at[dst], sem))
        copies.append(pltpu.async_copy(db_hbm.at[vidx.at[j]], dval.at[dst], sem))
    for c in copies:
        c.wait()

    @pl.loop(0, BPW, step=L)
    def _(i):
        s = pl.ds(i, L)
        out_v[s] = uval[s] + ival[s] + dval[s]

    pltpu.sync_copy(out_v, out_hbm.at[pl.ds(base, BPW)])


def kernel(users, items, values, user_bias, item_bias, data_bias):
    users = jnp.asarray(users, jnp.int32).reshape(NW * NCHUNK, CHUNK)
    items = jnp.asarray(items, jnp.int32).reshape(NW * NCHUNK, CHUNK)
    values = jnp.asarray(values, jnp.int32).reshape(NW * NCHUNK, CHUNK)
    return _three_bias_sum(
        users, items, values,
        user_bias.reshape(-1), item_bias.reshape(-1), data_bias.reshape(-1),
    )


# speedup vs baseline: 4.0114x; 1.8254x over previous
import functools

import jax
import jax.numpy as jnp
from jax import lax
from jax.experimental import pallas as pl
from jax.experimental.pallas import tpu as pltpu
from jax.experimental.pallas import tpu_sc as plsc

NC = 2
NS = 16
L = 16
NW = NC * NS

BATCH = 16384
BPW = BATCH // NW
CHUNK = 128
NCHUNK = BPW // CHUNK

_mesh = plsc.VectorSubcoreMesh(
    core_axis_name="c", subcore_axis_name="s", num_cores=NC, num_subcores=NS
)


@functools.partial(
    pl.kernel,
    out_type=jax.ShapeDtypeStruct((BATCH,), jnp.float32),
    mesh=_mesh,
    scratch_types=[
        pltpu.VMEM((NCHUNK, CHUNK), jnp.int32),
        pltpu.VMEM((NCHUNK, CHUNK), jnp.int32),
        pltpu.VMEM((NCHUNK, CHUNK), jnp.int32),
        pltpu.VMEM((BPW,), jnp.float32),
        pltpu.VMEM((BPW,), jnp.float32),
        pltpu.VMEM((BPW,), jnp.float32),
        pltpu.VMEM((BPW,), jnp.float32),
        pltpu.SemaphoreType.DMA,
    ],
)
def _three_bias_sum(users_hbm, items_hbm, values_hbm, ub_hbm, ib_hbm, db_hbm,
                    out_hbm, uidx, iidx, vidx, uval, ival, dval, out_v, sem):
    wid = lax.axis_index("s") * NC + lax.axis_index("c")
    base = wid * BPW

    row = wid * NCHUNK
    c_u = pltpu.async_copy(users_hbm.at[pl.ds(row, NCHUNK)], uidx, sem)
    c_i = pltpu.async_copy(items_hbm.at[pl.ds(row, NCHUNK)], iidx, sem)
    c_v = pltpu.async_copy(values_hbm.at[pl.ds(row, NCHUNK)], vidx, sem)
    c_u.wait()
    c_i.wait()
    c_v.wait()

    copies = []
    for j in range(NCHUNK):
        dst = pl.ds(j * CHUNK, CHUNK)
        copies.append(pltpu.async_copy(ub_hbm.at[uidx.at[j]], uval.at[dst], sem))
        copies.append(pltpu.async_copy(ib_hbm.at[iidx.at[j]], ival.at[dst], sem))
        copies.append(pltpu.async_copy(db_hbm.at[vidx.at[j]], dval.at[dst], sem))
    for c in copies:
        c.wait()

    @pl.loop(0, BPW, step=L)
    def _(i):
        s = pl.ds(i, L)
        out_v[s] = uval[s] + ival[s] + dval[s]

    pltpu.sync_copy(out_v, out_hbm.at[pl.ds(base, BPW)])


def _flatten_table(t):
    n = t.shape[0]
    n_pad = -n % 1024
    if n_pad:
        t = jnp.pad(t, ((0, n_pad), (0, 0)))
    return t.reshape(-1)


def kernel(users, items, values, user_bias, item_bias, data_bias):
    users = jnp.asarray(users, jnp.int32).reshape(NW * NCHUNK, CHUNK)
    items = jnp.asarray(items, jnp.int32).reshape(NW * NCHUNK, CHUNK)
    values = jnp.asarray(values, jnp.int32).reshape(NW * NCHUNK, CHUNK)
    return _three_bias_sum(
        users, items, values,
        _flatten_table(user_bias),
        _flatten_table(item_bias),
        _flatten_table(data_bias),
    )


# speedup vs baseline: 5.1607x; 1.2865x over previous
import dataclasses
import functools

import jax
import jax.numpy as jnp
from jax import lax
from jax.experimental import pallas as pl
from jax.experimental.pallas import tpu as pltpu
from jax.experimental.pallas import tpu_sc as plsc

NC = 2
NS = 16
L = 16
NW = NC * NS

BATCH = 16384
BPW = BATCH // NW
CHUNK = 128
NCHUNK = BPW // CHUNK

N_DATA_PAD = 1024

_mesh = plsc.VectorSubcoreMesh(
    core_axis_name="c", subcore_axis_name="s", num_cores=NC, num_subcores=NS
)

_cp = pltpu.CompilerParams()
if "needs_layout_passes" in pltpu.CompilerParams.__dataclass_fields__:
    _cp = dataclasses.replace(_cp, needs_layout_passes=False)


@functools.partial(
    pl.kernel,
    out_type=jax.ShapeDtypeStruct((BATCH,), jnp.float32),
    mesh=_mesh,
    scratch_types=[
        pltpu.VMEM((NCHUNK, CHUNK), jnp.int32),
        pltpu.VMEM((NCHUNK, CHUNK), jnp.int32),
        pltpu.VMEM((BPW,), jnp.float32),
        pltpu.VMEM((N_DATA_PAD,), jnp.float32),
        pltpu.VMEM((BPW,), jnp.float32),
        pltpu.SemaphoreType.DMA,
    ],
    compiler_params=_cp,
)
def _item_data_sum(items_hbm, values_hbm, ib_hbm, db_hbm, part_hbm,
                   iidx, vidx, ival, dtab, part_v, sem):
    wid = lax.axis_index("s") * NC + lax.axis_index("c")
    base = wid * BPW
    row = wid * NCHUNK

    c_i = pltpu.async_copy(items_hbm.at[pl.ds(row, NCHUNK)], iidx, sem)
    c_v = pltpu.async_copy(values_hbm.at[pl.ds(row, NCHUNK)], vidx, sem)
    c_d = pltpu.async_copy(db_hbm, dtab, sem)
    c_i.wait()
    c_v.wait()
    c_d.wait()

    copies = [
        pltpu.async_copy(ib_hbm.at[iidx.at[j]],
                         ival.at[pl.ds(j * CHUNK, CHUNK)], sem)
        for j in range(NCHUNK)
    ]
    for c in copies:
        c.wait()

    for j in range(NCHUNK):
        @pl.loop(0, CHUNK, step=L)
        def _(c0, j=j):
            s = pl.ds(j * CHUNK + c0, L)
            d = plsc.load_gather(dtab, [vidx[j, pl.ds(c0, L)]])
            part_v[s] = ival[s] + d

    pltpu.sync_copy(part_v, part_hbm.at[pl.ds(base, BPW)])


@functools.partial(
    pl.kernel,
    out_type=jax.ShapeDtypeStruct((BATCH,), jnp.float32),
    mesh=_mesh,
    scratch_types=[
        pltpu.VMEM((NCHUNK, CHUNK), jnp.int32),
        pltpu.VMEM((BPW,), jnp.float32),
        pltpu.VMEM((BPW,), jnp.float32),
        pltpu.SemaphoreType.DMA,
    ],
)
def _add_user(users_hbm, ub_hbm, part_hbm, out_hbm, uidx, uval, part_v, sem):
    wid = lax.axis_index("s") * NC + lax.axis_index("c")
    base = wid * BPW
    row = wid * NCHUNK

    c_u = pltpu.async_copy(users_hbm.at[pl.ds(row, NCHUNK)], uidx, sem)
    c_p = pltpu.async_copy(part_hbm.at[pl.ds(base, BPW)], part_v, sem)
    c_u.wait()
    c_p.wait()

    copies = [
        pltpu.async_copy(ub_hbm.at[uidx.at[j]],
                         uval.at[pl.ds(j * CHUNK, CHUNK)], sem)
        for j in range(NCHUNK)
    ]
    for c in copies:
        c.wait()

    @pl.loop(0, BPW, step=L)
    def _(i):
        s = pl.ds(i, L)
        part_v[s] = part_v[s] + uval[s]

    pltpu.sync_copy(part_v, out_hbm.at[pl.ds(base, BPW)])


def _flatten_table(t):
    n = t.shape[0]
    n_pad = -n % 1024
    if n_pad:
        t = jnp.pad(t, ((0, n_pad), (0, 0)))
    return t.reshape(-1)


def kernel(users, items, values, user_bias, item_bias, data_bias):
    users = jnp.asarray(users, jnp.int32).reshape(NW * NCHUNK, CHUNK)
    items = jnp.asarray(items, jnp.int32).reshape(NW * NCHUNK, CHUNK)
    values = jnp.asarray(values, jnp.int32).reshape(NW * NCHUNK, CHUNK)
    partial = _item_data_sum(
        items, values, _flatten_table(item_bias), _flatten_table(data_bias)
    )
    return _add_user(users, _flatten_table(user_bias), partial)
